# baseline (speedup 1.0000x reference)
import functools

import jax
import jax.numpy as jnp
from jax import lax
from jax.experimental import pallas as pl
from jax.experimental.pallas import tpu as pltpu
from jax.experimental.pallas import tpu_sc as plsc

D = 256
BAG = 4
NSUB = 16
NW = 32
CHUNK = 32
LANES = 16
NT_PAD = 1024
ET_PAD = 512


def _bag_sum(rows_v):
    @plsc.parallel_loop(0, CHUNK, step=1, unroll=2)
    def row_body(i):
        for d in range(D // LANES):
            sl = pl.ds(d * LANES, LANES)
            s = (rows_v[i, sl] + rows_v[CHUNK + i, sl]) + (
                rows_v[2 * CHUNK + i, sl] + rows_v[3 * CHUNK + i, sl])
            rows_v[i, sl] = s


def _embed_bag_phase(wid, idx_hbm, tab_hbm, out_hbm, idx_all, rows0,
                     rows1, sem0, sem1, rows_per_worker):
    base = wid * rows_per_worker
    npairs = rows_per_worker // (2 * CHUNK)
    nidx = rows_per_worker * BAG
    pltpu.sync_copy(idx_hbm.at[pl.ds(base * BAG, nidx)],
                    idx_all.at[pl.ds(0, nidx)])

    def fire(k, rows_v, sem):
        idx = idx_all.at[pl.ds(k * (CHUNK * BAG), CHUNK * BAG)]
        pltpu.async_copy(tab_hbm.at[idx], rows_v, sem)

    def finish(k, rows_v, sem):
        idx = idx_all.at[pl.ds(k * (CHUNK * BAG), CHUNK * BAG)]
        pltpu.make_async_copy(tab_hbm.at[idx], rows_v, sem).wait()
        _bag_sum(rows_v)
        pltpu.sync_copy(rows_v.at[pl.ds(0, CHUNK)],
                        out_hbm.at[pl.ds(base + k * CHUNK, CHUNK)])

    fire(0, rows0, sem0)

    def pair_body(p, carry):
        k = 2 * p
        fire(k + 1, rows1, sem1)
        finish(k, rows0, sem0)
        @pl.when(p < npairs - 1)
        def _():
            fire(k + 2, rows0, sem0)
        finish(k + 1, rows1, sem1)
        return carry

    lax.fori_loop(0, npairs, pair_body, 0)


def _make_kernel(nv_pad, ne_pad):
    mesh = plsc.VectorSubcoreMesh(core_axis_name="c", subcore_axis_name="s")

    @functools.partial(
        pl.kernel,
        mesh=mesh,
        out_type=[
            jax.ShapeDtypeStruct((nv_pad, D), jnp.float32),
            jax.ShapeDtypeStruct((ne_pad, D), jnp.float32),
        ],
        scratch_types=[
            pltpu.VMEM((20224,), jnp.int32),
            pltpu.VMEM((CHUNK * BAG, D), jnp.float32),
            pltpu.VMEM((CHUNK * BAG, D), jnp.float32),
            pltpu.SemaphoreType.DMA,
            pltpu.SemaphoreType.DMA,
        ],
    )
    def k(vidx_hbm, eidx_hbm, ntab_hbm, etab_hbm, outv_hbm, oute_hbm,
          idx_all, rows0, rows1, sem0, sem1):
        wid = lax.axis_index("s") * 2 + lax.axis_index("c")
        _embed_bag_phase(wid, vidx_hbm, ntab_hbm, outv_hbm, idx_all,
                         rows0, rows1, sem0, sem1, nv_pad // NW)
        _embed_bag_phase(wid, eidx_hbm, etab_hbm, oute_hbm, idx_all,
                         rows0, rows1, sem0, sem1, ne_pad // NW)

    return k


def _prep_rows(idx, mult):
    n = idx.shape[0]
    n_pad = ((n + mult - 1) // mult) * mult
    idx = jnp.pad(idx, ((0, n_pad - n), (0, 0)))
    idx = idx.reshape(n_pad // CHUNK, CHUNK, BAG).transpose(0, 2, 1)
    return idx.reshape(-1), n_pad


def kernel(V, E, node_table, edge_table):
    n_nodes = V.shape[0]
    n_edges = E.shape[0]
    v_flat, nv_pad = _prep_rows(V, NW * CHUNK * 2)
    e_flat, ne_pad = _prep_rows(E, NW * CHUNK * 2)
    ntab = jnp.pad(node_table, ((0, NT_PAD - node_table.shape[0]), (0, 0)))
    v_emb, e_emb = _make_kernel(nv_pad, ne_pad)(
        v_flat, e_flat, ntab, edge_table)
    return (v_emb[:n_nodes], e_emb[:n_edges])
